# baseline (speedup 1.0000x reference)
import numpy as np
import jax
import jax.numpy as jnp
from jax.experimental import pallas as pl
from jax.experimental.pallas import tpu as pltpu

N = 1024
NORI = 6
HID = 64
OUT = 32
NSEG = 32
ROWS_PER_SEG = N // NSEG
SEG_CAP = 2048
ECAP = NSEG * SEG_CAP
MSG_BLK = 512
RADIUS = 2.0


def _fib_sphere(n):
    i = np.arange(n, dtype=np.float64) + 0.5
    phi = np.arccos(1.0 - 2.0 * i / n)
    theta = np.pi * (1.0 + 5.0 ** 0.5) * i
    return np.stack([np.cos(theta) * np.sin(phi), np.sin(theta) * np.sin(phi),
                     np.cos(phi)], axis=-1).astype(np.float32)

_ORI = _fib_sphere(NORI)
_B3 = np.arccos(np.clip((_ORI @ _ORI.T), -1.0 + 1e-6, 1.0 - 1e-6))
_B3 = _B3.reshape(NORI * NORI, 1).astype(np.float32)


def _silu(v):
    return v / (1.0 + jnp.exp(-v))


def _kx_body(counts_ref, dx_ref, dy_ref, dz_ref, bx_ref, w1_ref, b1_ref,
             w2_ref, b2_ref, out_ref):
    seg = pl.program_id(0)
    cnt = counts_ref[seg]
    dx = dx_ref[...]
    dy = dy_ref[...]
    dz = dz_ref[...]
    d2 = dx * dx + dy * dy + dz * dz
    dist = jnp.sqrt(d2 + 1e-12)
    mw = 0.5 * (jnp.cos((np.pi / RADIUS) * dist) + 1.0)
    eidx = jax.lax.broadcasted_iota(jnp.int32, (SEG_CAP, 1), 0)
    validf = jnp.where(eidx < cnt, 1.0, 0.0)
    wt = mw * validf
    b0 = bx_ref[0:1, :]
    b1r = bx_ref[1:2, :]
    for a in range(NORI):
        ox, oy, oz = (float(_ORI[a, 0]), float(_ORI[a, 1]), float(_ORI[a, 2]))
        a1 = dx * ox + dy * oy + dz * oz
        px = dx - a1 * ox
        py = dy - a1 * oy
        pz = dz - a1 * oz
        a2 = jnp.sqrt(px * px + py * py + pz * pz + 1e-12)
        z = (2.0 * np.pi) * (a1 * b0 + a2 * b1r)
        rff = jnp.concatenate([jnp.cos(z), jnp.sin(z)], axis=1)
        h = _silu(jnp.dot(rff, w1_ref[...],
                          preferred_element_type=jnp.float32) + b1_ref[...])
        h = _silu(jnp.dot(h, w2_ref[...],
                          preferred_element_type=jnp.float32) + b2_ref[...])
        out_ref[:, a, :] = h * wt


def _run_kx(counts, dxc, dyc, dzc, p):
    full = lambda shape: pl.BlockSpec(shape, lambda i, c: tuple(0 for _ in shape))
    return pl.pallas_call(
        _kx_body,
        grid_spec=pltpu.PrefetchScalarGridSpec(
            num_scalar_prefetch=1,
            grid=(NSEG,),
            in_specs=[
                pl.BlockSpec((SEG_CAP, 1), lambda i, c: (i, 0)),
                pl.BlockSpec((SEG_CAP, 1), lambda i, c: (i, 0)),
                pl.BlockSpec((SEG_CAP, 1), lambda i, c: (i, 0)),
                full((2, HID // 2)),
                full((HID, HID)), full((1, HID)),
                full((HID, HID)), full((1, HID)),
            ],
            out_specs=pl.BlockSpec((SEG_CAP, NORI, HID), lambda i, c: (i, 0, 0)),
        ),
        out_shape=jax.ShapeDtypeStruct((ECAP, NORI, HID), jnp.float32),
    )(counts, dxc, dyc, dzc, p["B_x"],
      p["ex1"]["W"], p["ex1"]["b"].reshape(1, -1),
      p["ex2"]["W"], p["ex2"]["b"].reshape(1, -1))


def _kinit_body(x_ref, e1w, e1b, e2w, e2b, m0w, m0b, b3_ref, br_ref,
                r1w, r1b, r2w, r2b, msrc_ref, r_ref):
    h = _silu(jnp.dot(x_ref[...], e1w[...],
                      preferred_element_type=jnp.float32) + e1b[...])
    h = jnp.dot(h, e2w[...], preferred_element_type=jnp.float32) + e2b[...]
    m0 = jnp.dot(h, m0w[...], preferred_element_type=jnp.float32) + m0b[...]
    for a in range(NORI):
        msrc_ref[:, a, :] = m0
    z = (2.0 * np.pi) * (b3_ref[...] * br_ref[...])
    rff = jnp.concatenate([jnp.cos(z), jnp.sin(z)], axis=1)
    r = _silu(jnp.dot(rff, r1w[...],
                      preferred_element_type=jnp.float32) + r1b[...])
    r = _silu(jnp.dot(r, r2w[...],
                      preferred_element_type=jnp.float32) + r2b[...])
    r_ref[...] = r


def _run_kinit(x, p):
    l0 = p["layers"][0]
    return pl.pallas_call(
        _kinit_body,
        out_shape=[jax.ShapeDtypeStruct((N, NORI, HID), jnp.float32),
                   jax.ShapeDtypeStruct((NORI * NORI, HID), jnp.float32)],
    )(x, p["emb1"]["W"], p["emb1"]["b"].reshape(1, -1),
      p["emb2"]["W"], p["emb2"]["b"].reshape(1, -1),
      l0["msg"]["W"], l0["msg"]["b"].reshape(1, -1),
      jnp.asarray(_B3), p["B_R"],
      p["eR1"]["W"], p["eR1"]["b"].reshape(1, -1),
      p["eR2"]["W"], p["eR2"]["b"].reshape(1, -1))


def _kmsg_body(counts_ref, src_ref, dst_ref, xt_ref, msrc_ref, agg_ref):
    step = pl.program_id(0)
    seg = step // (SEG_CAP // MSG_BLK)
    sub = step % (SEG_CAP // MSG_BLK)

    @pl.when(step == 0)
    def _():
        agg_ref[...] = jnp.zeros_like(agg_ref)

    @pl.when(sub * MSG_BLK < counts_ref[seg])
    def _():
        srcb = src_ref[0, 0, :]
        dstb = dst_ref[0, 0, :]
        nidx = jax.lax.broadcasted_iota(jnp.int32, (MSG_BLK, N), 1)
        ohs = jnp.where(srcb[:, None] == nidx, 1.0, 0.0)
        ohd = jnp.where(dstb[:, None] == nidx, 1.0, 0.0)
        for a in range(NORI):
            g = jnp.dot(ohs, msrc_ref[:, a, :],
                        preferred_element_type=jnp.float32)
            msg = xt_ref[:, a, :] * g
            agg_ref[:, a, :] += jax.lax.dot_general(
                ohd, msg, (((0,), (0,)), ((), ())),
                preferred_element_type=jnp.float32)


def _run_kmsg(counts, src3, dst3, xt, msrc):
    nsub = SEG_CAP // MSG_BLK
    return pl.pallas_call(
        _kmsg_body,
        grid_spec=pltpu.PrefetchScalarGridSpec(
            num_scalar_prefetch=1,
            grid=(NSEG * nsub,),
            in_specs=[
                pl.BlockSpec((1, 1, MSG_BLK), lambda i, c: (i, 0, 0)),
                pl.BlockSpec((1, 1, MSG_BLK), lambda i, c: (i, 0, 0)),
                pl.BlockSpec((MSG_BLK, NORI, HID), lambda i, c: (i, 0, 0)),
                pl.BlockSpec((N, NORI, HID), lambda i, c: (0, 0, 0)),
            ],
            out_specs=pl.BlockSpec((N, NORI, HID), lambda i, c: (0, 0, 0)),
        ),
        out_shape=jax.ShapeDtypeStruct((N, NORI, HID), jnp.float32),
    )(counts, src3, dst3, xt, msrc)


def _make_kupd(i):
    last = (i == 2)
    first = (i == 0)

    def body(*refs):
        it = iter(refs)
        agg_ref = next(it)
        h_prev_ref = None if first else next(it)
        pred_ref = next(it)
        r_ref = None if last else next(it)
        u1w, u1b, u2w, u2b = next(it), next(it), next(it), next(it)
        p1w, p1b, p2w, p2b = next(it), next(it), next(it), next(it)
        if not last:
            mw_, mb_ = next(it), next(it)
        outs = list(it)

        agg = agg_ref[...].reshape(N * NORI, HID)
        t = _silu(jnp.dot(agg, u1w[...],
                          preferred_element_type=jnp.float32) + u1b[...])
        upd = jnp.dot(t, u2w[...], preferred_element_type=jnp.float32) + u2b[...]
        h = upd if first else h_prev_ref[...] + upd
        h3 = h.reshape(N, NORI, HID)
        hrd = jnp.sum(h3, axis=1) * (1.0 / NORI)
        tp = _silu(jnp.dot(hrd, p1w[...],
                           preferred_element_type=jnp.float32) + p1b[...])
        pred = pred_ref[...] + jnp.dot(tp, p2w[...],
                                       preferred_element_type=jnp.float32) + p2b[...]
        if last:
            outs[0][...] = jnp.sum(pred, axis=0, keepdims=True)
        else:
            h_out, msrc_out, pred_out = outs
            h_out[...] = h
            pred_out[...] = pred
            hm = jnp.dot(h, mw_[...],
                         preferred_element_type=jnp.float32) + mb_[...]
            hm3 = hm.reshape(N, NORI, HID)
            for b in range(NORI):
                acc = hm3[:, 0, :] * r_ref[0 * NORI + b, :][None, :]
                for a in range(1, NORI):
                    acc = acc + hm3[:, a, :] * r_ref[a * NORI + b, :][None, :]
                msrc_out[:, b, :] = acc

    return body


def _run_kupd(i, agg, h_prev, pred, r, p):
    last = (i == 2)
    lp = p["layers"][i]
    pp = p["post"][i]
    args = [agg]
    if i != 0:
        args.append(h_prev)
    args.append(pred)
    if not last:
        args.append(r)
    args += [lp["u1"]["W"], lp["u1"]["b"].reshape(1, -1),
             lp["u2"]["W"], lp["u2"]["b"].reshape(1, -1),
             pp["p1"]["W"], pp["p1"]["b"].reshape(1, -1),
             pp["p2"]["W"], pp["p2"]["b"].reshape(1, -1)]
    if not last:
        ln = p["layers"][i + 1]
        args += [ln["msg"]["W"], ln["msg"]["b"].reshape(1, -1)]
    if last:
        out_shape = [jax.ShapeDtypeStruct((1, OUT), jnp.float32)]
    else:
        out_shape = [jax.ShapeDtypeStruct((N * NORI, HID), jnp.float32),
                     jax.ShapeDtypeStruct((N, NORI, HID), jnp.float32),
                     jax.ShapeDtypeStruct((N, OUT), jnp.float32)]
    return pl.pallas_call(_make_kupd(i), out_shape=out_shape)(*args)


def _build_edges(pos):
    px, py, pz = pos[:, 0], pos[:, 1], pos[:, 2]
    d2 = ((px[:, None] - px[None, :]) ** 2 + (py[:, None] - py[None, :]) ** 2
          + (pz[:, None] - pz[None, :]) ** 2)
    m = d2 <= RADIUS * RADIUS
    mband = m.reshape(NSEG, ROWS_PER_SEG * N)
    counts = mband.sum(axis=1).astype(jnp.int32)
    iota = jnp.arange(ROWS_PER_SEG * N, dtype=jnp.int32)
    big = jnp.int32(1 << 30)
    ids = jnp.sort(jnp.where(mband, iota[None, :], big), axis=1)[:, :SEG_CAP]
    ids = jnp.minimum(ids, ROWS_PER_SEG * N - 1)
    src = (ids // N) + (jnp.arange(NSEG, dtype=jnp.int32) * ROWS_PER_SEG)[:, None]
    dst = ids % N
    src = src.reshape(-1)
    dst = dst.reshape(-1)
    diffx = (px[src] - px[dst]).reshape(ECAP, 1)
    diffy = (py[src] - py[dst]).reshape(ECAP, 1)
    diffz = (pz[src] - pz[dst]).reshape(ECAP, 1)
    nsub = SEG_CAP // MSG_BLK
    src3 = src.reshape(NSEG * nsub, 1, MSG_BLK)
    dst3 = dst.reshape(NSEG * nsub, 1, MSG_BLK)
    return counts, src3, dst3, diffx, diffy, diffz


def kernel(pos, x, batch, params):
    counts, src3, dst3, dxc, dyc, dzc = _build_edges(pos)
    xt = _run_kx(counts, dxc, dyc, dzc, params)
    msrc, r = _run_kinit(x, params)
    h = None
    pred = jnp.zeros((N, OUT), jnp.float32)
    for i in range(3):
        agg = _run_kmsg(counts, src3, dst3, xt, msrc)
        if i < 2:
            h, msrc, pred = _run_kupd(i, agg, h, pred, r, params)
        else:
            (out,) = _run_kupd(i, agg, h, pred, None, params)
    return out


# speedup vs baseline: 2.2212x; 2.2212x over previous
import functools
import numpy as np
import jax
import jax.numpy as jnp
from jax import lax
from jax.experimental import pallas as pl
from jax.experimental.pallas import tpu as pltpu
from jax.experimental.pallas import tpu_sc as plsc

N = 1024
NORI = 6
HID = 64
OUT = 32
NSEG = 32
ROWS_PER_SEG = N // NSEG
SEG_CAP = 2048
ECAP = NSEG * SEG_CAP
MSG_BLK = 512
RADIUS = 2.0


def _fib_sphere(n):
    i = np.arange(n, dtype=np.float64) + 0.5
    phi = np.arccos(1.0 - 2.0 * i / n)
    theta = np.pi * (1.0 + 5.0 ** 0.5) * i
    return np.stack([np.cos(theta) * np.sin(phi), np.sin(theta) * np.sin(phi),
                     np.cos(phi)], axis=-1).astype(np.float32)

_ORI = _fib_sphere(NORI)
_B3 = np.arccos(np.clip((_ORI @ _ORI.T), -1.0 + 1e-6, 1.0 - 1e-6))
_B3 = _B3.reshape(NORI * NORI, 1).astype(np.float32)


def _silu(v):
    return v / (1.0 + jnp.exp(-v))


def _kx_body(counts_ref, dx_ref, dy_ref, dz_ref, bx_ref, w1_ref, b1_ref,
             w2_ref, b2_ref, out_ref):
    seg = pl.program_id(0)
    cnt = counts_ref[seg]
    dx = dx_ref[...]
    dy = dy_ref[...]
    dz = dz_ref[...]
    d2 = dx * dx + dy * dy + dz * dz
    dist = jnp.sqrt(d2 + 1e-12)
    mw = 0.5 * (jnp.cos((np.pi / RADIUS) * dist) + 1.0)
    eidx = jax.lax.broadcasted_iota(jnp.int32, (SEG_CAP, 1), 0)
    valid = eidx < cnt
    wt = jnp.where(valid, mw, 0.0)
    b0 = bx_ref[0:1, :]
    b1r = bx_ref[1:2, :]
    for a in range(NORI):
        ox, oy, oz = (float(_ORI[a, 0]), float(_ORI[a, 1]), float(_ORI[a, 2]))
        a1 = dx * ox + dy * oy + dz * oz
        px = dx - a1 * ox
        py = dy - a1 * oy
        pz = dz - a1 * oz
        a2 = jnp.sqrt(px * px + py * py + pz * pz + 1e-12)
        z = (2.0 * np.pi) * (a1 * b0 + a2 * b1r)
        rff = jnp.concatenate([jnp.cos(z), jnp.sin(z)], axis=1)
        h = _silu(jnp.dot(rff, w1_ref[...],
                          preferred_element_type=jnp.float32) + b1_ref[...])
        h = _silu(jnp.dot(h, w2_ref[...],
                          preferred_element_type=jnp.float32) + b2_ref[...])
        out_ref[:, a, :] = jnp.where(valid, h * wt, 0.0)


def _run_kx(counts, dxc, dyc, dzc, p):
    full = lambda shape: pl.BlockSpec(shape, lambda i, c: tuple(0 for _ in shape))
    return pl.pallas_call(
        _kx_body,
        grid_spec=pltpu.PrefetchScalarGridSpec(
            num_scalar_prefetch=1,
            grid=(NSEG,),
            in_specs=[
                pl.BlockSpec((SEG_CAP, 1), lambda i, c: (i, 0)),
                pl.BlockSpec((SEG_CAP, 1), lambda i, c: (i, 0)),
                pl.BlockSpec((SEG_CAP, 1), lambda i, c: (i, 0)),
                full((2, HID // 2)),
                full((HID, HID)), full((1, HID)),
                full((HID, HID)), full((1, HID)),
            ],
            out_specs=pl.BlockSpec((SEG_CAP, NORI, HID), lambda i, c: (i, 0, 0)),
        ),
        out_shape=jax.ShapeDtypeStruct((ECAP, NORI, HID), jnp.float32),
    )(counts, dxc, dyc, dzc, p["B_x"],
      p["ex1"]["W"], p["ex1"]["b"].reshape(1, -1),
      p["ex2"]["W"], p["ex2"]["b"].reshape(1, -1))


def _kinit_body(x_ref, e1w, e1b, e2w, e2b, m0w, m0b, b3_ref, br_ref,
                r1w, r1b, r2w, r2b, msrc_ref, r_ref):
    h = _silu(jnp.dot(x_ref[...], e1w[...],
                      preferred_element_type=jnp.float32) + e1b[...])
    h = jnp.dot(h, e2w[...], preferred_element_type=jnp.float32) + e2b[...]
    m0 = jnp.dot(h, m0w[...], preferred_element_type=jnp.float32) + m0b[...]
    for a in range(NORI):
        msrc_ref[:, a, :] = m0
    z = (2.0 * np.pi) * (b3_ref[...] * br_ref[...])
    rff = jnp.concatenate([jnp.cos(z), jnp.sin(z)], axis=1)
    r = _silu(jnp.dot(rff, r1w[...],
                      preferred_element_type=jnp.float32) + r1b[...])
    r = _silu(jnp.dot(r, r2w[...],
                      preferred_element_type=jnp.float32) + r2b[...])
    r_ref[...] = r


def _run_kinit(x, p):
    l0 = p["layers"][0]
    return pl.pallas_call(
        _kinit_body,
        out_shape=[jax.ShapeDtypeStruct((N, NORI, HID), jnp.float32),
                   jax.ShapeDtypeStruct((NORI * NORI, HID), jnp.float32)],
    )(x, p["emb1"]["W"], p["emb1"]["b"].reshape(1, -1),
      p["emb2"]["W"], p["emb2"]["b"].reshape(1, -1),
      l0["msg"]["W"], l0["msg"]["b"].reshape(1, -1),
      jnp.asarray(_B3), p["B_R"],
      p["eR1"]["W"], p["eR1"]["b"].reshape(1, -1),
      p["eR2"]["W"], p["eR2"]["b"].reshape(1, -1))


NC = 2
NS = 16
ROWS_PER_TILE = N // NS
BE = 16


def _sc_mesh():
    return plsc.VectorSubcoreMesh(core_axis_name="c", subcore_axis_name="s",
                                  num_cores=NC, num_subcores=NS)


def _run_compact(posx, posy, posz):
    @functools.partial(
        pl.kernel,
        out_type=[jax.ShapeDtypeStruct((NSEG, 16), jnp.int32),
                  jax.ShapeDtypeStruct((ECAP,), jnp.int32),
                  jax.ShapeDtypeStruct((ECAP,), jnp.int32),
                  jax.ShapeDtypeStruct((ECAP,), jnp.float32),
                  jax.ShapeDtypeStruct((ECAP,), jnp.float32),
                  jax.ShapeDtypeStruct((ECAP,), jnp.float32)],
        mesh=_sc_mesh(),
        scratch_types=[pltpu.VMEM((N + 16,), jnp.float32),
                       pltpu.VMEM((N + 16,), jnp.float32),
                       pltpu.VMEM((N + 16,), jnp.float32),
                       pltpu.VMEM((SEG_CAP,), jnp.int32),
                       pltpu.VMEM((SEG_CAP,), jnp.int32),
                       pltpu.VMEM((SEG_CAP,), jnp.float32),
                       pltpu.VMEM((SEG_CAP,), jnp.float32),
                       pltpu.VMEM((SEG_CAP,), jnp.float32),
                       pltpu.VMEM((16,), jnp.int32)],
    )
    def k(px_h, py_h, pz_h, cnt_h, src_h, dst_h, dx_h, dy_h, dz_h,
          px_v, py_v, pz_v, src_b, dst_b, dxb, dyb, dzb, cnt_v):
        wid = lax.axis_index("s") * NC + lax.axis_index("c")
        pltpu.sync_copy(px_h, px_v.at[pl.ds(0, N)])
        pltpu.sync_copy(py_h, py_v.at[pl.ds(0, N)])
        pltpu.sync_copy(pz_h, pz_v.at[pl.ds(0, N)])
        z16i = jnp.zeros((16,), jnp.int32)
        z16f = jnp.zeros((16,), jnp.float32)

        def zf(i, carry):
            sl = pl.ds(i * 16, 16)
            src_b[sl] = z16i
            dst_b[sl] = z16i
            dxb[sl] = z16f
            dyb[sl] = z16f
            dzb[sl] = z16f
            return carry

        lax.fori_loop(0, SEG_CAP // 16, zf, 0)

        def outer(row, off):
            sg = wid * ROWS_PER_SEG + row
            sx = jnp.full((16,), px_v[pl.ds(sg, 16)][0])
            sy = jnp.full((16,), py_v[pl.ds(sg, 16)][0])
            sz = jnp.full((16,), pz_v[pl.ds(sg, 16)][0])

            def inner(ch, off):
                base = ch * 16
                dxv = sx - px_v[pl.ds(base, 16)]
                dyv = sy - py_v[pl.ds(base, 16)]
                dzv = sz - pz_v[pl.ds(base, 16)]
                d2 = dxv * dxv + dyv * dyv + dzv * dzv
                mi = jnp.where(d2 <= RADIUS * RADIUS, 1, 0)
                for j in range(16):
                    mj = mi[j]
                    ofu = jnp.minimum(off, SEG_CAP - 16)

                    @pl.when(mj == 1)
                    def _(j=j, ofu=ofu, dxv=dxv, dyv=dyv, dzv=dzv, base=base):
                        sl = pl.ds(ofu, 16)
                        dst_b[sl] = jnp.full((16,), base + j, jnp.int32)
                        src_b[sl] = jnp.full((16,), sg, jnp.int32)
                        dxb[sl] = jnp.full((16,), dxv[j])
                        dyb[sl] = jnp.full((16,), dyv[j])
                        dzb[sl] = jnp.full((16,), dzv[j])

                    off = off + mj
                return off

            return lax.fori_loop(0, N // 16, inner, off)

        off = lax.fori_loop(0, ROWS_PER_SEG, outer, jnp.int32(0))
        off = jnp.minimum(off, SEG_CAP)
        base = wid * SEG_CAP
        pltpu.sync_copy(src_b, src_h.at[pl.ds(base, SEG_CAP)])
        pltpu.sync_copy(dst_b, dst_h.at[pl.ds(base, SEG_CAP)])
        pltpu.sync_copy(dxb, dx_h.at[pl.ds(base, SEG_CAP)])
        pltpu.sync_copy(dyb, dy_h.at[pl.ds(base, SEG_CAP)])
        pltpu.sync_copy(dzb, dz_h.at[pl.ds(base, SEG_CAP)])
        cnt_v[...] = jnp.full((16,), off, jnp.int32)
        pltpu.sync_copy(cnt_v, cnt_h.at[wid])

    return k(posx, posy, posz)


def _run_msg_sc(counts2d, src, dst, xt2, msrc2):
    F = NORI * HID
    NR = F // 128
    RPT = (N * NR) // NS

    @functools.partial(
        pl.kernel,
        out_type=jax.ShapeDtypeStruct((NC, N * NR, 128), jnp.float32),
        mesh=_sc_mesh(),
        scratch_types=[pltpu.VMEM((16,), jnp.int32),
                       pltpu.VMEM((BE,), jnp.int32),
                       pltpu.VMEM((BE,), jnp.int32),
                       pltpu.VMEM((NR * BE,), jnp.int32),
                       pltpu.VMEM((BE, F), jnp.float32),
                       pltpu.VMEM((BE, F), jnp.float32),
                       pltpu.VMEM((NR * BE, 128), jnp.float32),
                       pltpu.VMEM((RPT, 128), jnp.float32),
                       pltpu.VMEM_SHARED((N * NR, 128), jnp.float32),
                       pltpu.SemaphoreType.DMA],
    )
    def k(cnt_h, src_h, dst_h, xt_h, ms_h, out_h,
          cnt_v, sidx, didx, idx3, xbuf, mbuf, cbuf, zbuf, agg_sh, sem):
        cid = lax.axis_index("c")
        sid = lax.axis_index("s")
        wid = sid * NC + cid

        def zf2(i, carry):
            r = i // 8
            c2 = (i % 8) * 16
            zbuf[r, pl.ds(c2, 16)] = jnp.zeros((16,), jnp.float32)
            return carry

        lax.fori_loop(0, RPT * 8, zf2, 0)
        pltpu.sync_copy(zbuf, agg_sh.at[pl.ds(sid * RPT, RPT)])
        plsc.subcore_barrier()

        pltpu.sync_copy(cnt_h.at[wid], cnt_v)
        cnt = cnt_v[...][0]
        nblk = (cnt + BE - 1) // BE

        def blk(b, carry):
            base = wid * SEG_CAP + b * BE
            pltpu.sync_copy(src_h.at[pl.ds(base, BE)], sidx)
            pltpu.sync_copy(dst_h.at[pl.ds(base, BE)], didx)
            pltpu.sync_copy(xt_h.at[pl.ds(base, BE)], xbuf)
            pltpu.async_copy(ms_h.at[sidx], mbuf, sem).wait()
            dvec = didx[...]
            for kk in range(NR):
                idx3[pl.ds(kk * BE, BE)] = dvec * NR + kk

            def mul(k2, carry2):
                e = k2 // (F // 16)
                cc = k2 % (F // 16)
                kk = cc // 8
                c2 = (cc % 8) * 16
                cbuf[kk * BE + e, pl.ds(c2, 16)] = (
                    xbuf[e, pl.ds(cc * 16, 16)] * mbuf[e, pl.ds(cc * 16, 16)])
                return carry2

            lax.fori_loop(0, BE * (F // 16), mul, 0)
            pltpu.sync_copy(cbuf, agg_sh.at[idx3], add=True)
            return carry

        lax.fori_loop(0, nblk, blk, 0)
        plsc.subcore_barrier()
        pltpu.sync_copy(agg_sh.at[pl.ds(sid * RPT, RPT)],
                        out_h.at[cid, pl.ds(sid * RPT, RPT)])

    return k(counts2d, src, dst, xt2, msrc2)


def _make_kupd(i):
    last = (i == 2)
    first = (i == 0)

    def body(*refs):
        it = iter(refs)
        agg_ref = next(it)
        h_prev_ref = None if first else next(it)
        pred_ref = next(it)
        r_ref = None if last else next(it)
        u1w, u1b, u2w, u2b = next(it), next(it), next(it), next(it)
        p1w, p1b, p2w, p2b = next(it), next(it), next(it), next(it)
        if not last:
            mw_, mb_ = next(it), next(it)
        outs = list(it)

        agg = jnp.sum(agg_ref[...], axis=0)
        t = _silu(jnp.dot(agg, u1w[...],
                          preferred_element_type=jnp.float32) + u1b[...])
        upd = jnp.dot(t, u2w[...], preferred_element_type=jnp.float32) + u2b[...]
        h = upd if first else h_prev_ref[...] + upd
        h3 = h.reshape(N, NORI, HID)
        hrd = jnp.sum(h3, axis=1) * (1.0 / NORI)
        tp = _silu(jnp.dot(hrd, p1w[...],
                           preferred_element_type=jnp.float32) + p1b[...])
        pred = pred_ref[...] + jnp.dot(tp, p2w[...],
                                       preferred_element_type=jnp.float32) + p2b[...]
        if last:
            outs[0][...] = jnp.sum(pred, axis=0, keepdims=True)
        else:
            h_out, msrc_out, pred_out = outs
            h_out[...] = h
            pred_out[...] = pred
            hm = jnp.dot(h, mw_[...],
                         preferred_element_type=jnp.float32) + mb_[...]
            hm3 = hm.reshape(N, NORI, HID)
            for b in range(NORI):
                acc = hm3[:, 0, :] * r_ref[0 * NORI + b, :][None, :]
                for a in range(1, NORI):
                    acc = acc + hm3[:, a, :] * r_ref[a * NORI + b, :][None, :]
                msrc_out[:, b, :] = acc

    return body


def _run_kupd(i, agg, h_prev, pred, r, p):
    last = (i == 2)
    lp = p["layers"][i]
    pp = p["post"][i]
    args = [agg]
    if i != 0:
        args.append(h_prev)
    args.append(pred)
    if not last:
        args.append(r)
    args += [lp["u1"]["W"], lp["u1"]["b"].reshape(1, -1),
             lp["u2"]["W"], lp["u2"]["b"].reshape(1, -1),
             pp["p1"]["W"], pp["p1"]["b"].reshape(1, -1),
             pp["p2"]["W"], pp["p2"]["b"].reshape(1, -1)]
    if not last:
        ln = p["layers"][i + 1]
        args += [ln["msg"]["W"], ln["msg"]["b"].reshape(1, -1)]
    if last:
        out_shape = [jax.ShapeDtypeStruct((1, OUT), jnp.float32)]
    else:
        out_shape = [jax.ShapeDtypeStruct((N * NORI, HID), jnp.float32),
                     jax.ShapeDtypeStruct((N, NORI, HID), jnp.float32),
                     jax.ShapeDtypeStruct((N, OUT), jnp.float32)]
    return pl.pallas_call(_make_kupd(i), out_shape=out_shape)(*args)


def kernel(pos, x, batch, params):
    px = pos[:, 0]
    py = pos[:, 1]
    pz = pos[:, 2]
    counts2d, src, dst, dxe, dye, dze = _run_compact(px, py, pz)
    counts = counts2d[:, 0]
    xt = _run_kx(counts, dxe.reshape(ECAP, 1), dye.reshape(ECAP, 1),
                 dze.reshape(ECAP, 1), params)
    xt2 = xt.reshape(ECAP, NORI * HID)
    msrc, r = _run_kinit(x, params)
    h = None
    pred = jnp.zeros((N, OUT), jnp.float32)
    for i in range(3):
        aggp = _run_msg_sc(counts2d, src, dst, xt2, msrc.reshape(N, NORI * HID))
        agg = aggp.reshape(NC, N * NORI, HID)
        if i < 2:
            h, msrc, pred = _run_kupd(i, agg, h, pred, r, params)
        else:
            (out,) = _run_kupd(i, agg, h, pred, None, params)
    return out


# speedup vs baseline: 3.6243x; 1.6317x over previous
import functools
import numpy as np
import jax
import jax.numpy as jnp
from jax import lax
from jax.experimental import pallas as pl
from jax.experimental.pallas import tpu as pltpu
from jax.experimental.pallas import tpu_sc as plsc

N = 1024
NORI = 6
HID = 64
OUT = 32
NSEG = 32
ROWS_PER_SEG = N // NSEG
SEG_CAP = 2048
ECAP = NSEG * SEG_CAP
MSG_BLK = 512
RADIUS = 2.0


def _fib_sphere(n):
    i = np.arange(n, dtype=np.float64) + 0.5
    phi = np.arccos(1.0 - 2.0 * i / n)
    theta = np.pi * (1.0 + 5.0 ** 0.5) * i
    return np.stack([np.cos(theta) * np.sin(phi), np.sin(theta) * np.sin(phi),
                     np.cos(phi)], axis=-1).astype(np.float32)

_ORI = _fib_sphere(NORI)
_B3 = np.arccos(np.clip((_ORI @ _ORI.T), -1.0 + 1e-6, 1.0 - 1e-6))
_B3 = _B3.reshape(NORI * NORI, 1).astype(np.float32)


def _silu(v):
    return v / (1.0 + jnp.exp(-v))


XBLK = 256
XSUB = SEG_CAP // XBLK


def _kx_body(counts_ref, dx_ref, dy_ref, dz_ref, bx_ref, w1_ref, b1_ref,
             w2_ref, b2_ref, out_ref):
    i = pl.program_id(0)
    seg = i // XSUB
    base = (i % XSUB) * XBLK
    cnt = counts_ref[seg]

    @pl.when(base < cnt)
    def _():
        dx = dx_ref[...]
        dy = dy_ref[...]
        dz = dz_ref[...]
        d2 = dx * dx + dy * dy + dz * dz
        dist = jnp.sqrt(d2 + 1e-12)
        mw = 0.5 * (jnp.cos((np.pi / RADIUS) * dist) + 1.0)
        eidx = jax.lax.broadcasted_iota(jnp.int32, (XBLK, 1), 0) + base
        valid = eidx < cnt
        wt = jnp.where(valid, mw, 0.0)
        b0 = bx_ref[0:1, :]
        b1r = bx_ref[1:2, :]
        for a in range(NORI):
            ox, oy, oz = (float(_ORI[a, 0]), float(_ORI[a, 1]),
                          float(_ORI[a, 2]))
            a1 = dx * ox + dy * oy + dz * oz
            px = dx - a1 * ox
            py = dy - a1 * oy
            pz = dz - a1 * oz
            a2 = jnp.sqrt(px * px + py * py + pz * pz + 1e-12)
            z = (2.0 * np.pi) * (a1 * b0 + a2 * b1r)
            rff = jnp.concatenate([jnp.cos(z), jnp.sin(z)], axis=1)
            h = _silu(jnp.dot(rff, w1_ref[...],
                              preferred_element_type=jnp.float32) + b1_ref[...])
            h = _silu(jnp.dot(h, w2_ref[...],
                              preferred_element_type=jnp.float32) + b2_ref[...])
            out_ref[:, a, :] = jnp.where(valid, h * wt, 0.0)


def _run_kx(counts, dxc, dyc, dzc, p):
    full = lambda shape: pl.BlockSpec(shape, lambda i, c: tuple(0 for _ in shape))
    return pl.pallas_call(
        _kx_body,
        grid_spec=pltpu.PrefetchScalarGridSpec(
            num_scalar_prefetch=1,
            grid=(NSEG * XSUB,),
            in_specs=[
                pl.BlockSpec((XBLK, 1), lambda i, c: (i, 0)),
                pl.BlockSpec((XBLK, 1), lambda i, c: (i, 0)),
                pl.BlockSpec((XBLK, 1), lambda i, c: (i, 0)),
                full((2, HID // 2)),
                full((HID, HID)), full((1, HID)),
                full((HID, HID)), full((1, HID)),
            ],
            out_specs=pl.BlockSpec((XBLK, NORI, HID), lambda i, c: (i, 0, 0)),
        ),
        out_shape=jax.ShapeDtypeStruct((ECAP, NORI, HID), jnp.float32),
    )(counts, dxc, dyc, dzc, p["B_x"],
      p["ex1"]["W"], p["ex1"]["b"].reshape(1, -1),
      p["ex2"]["W"], p["ex2"]["b"].reshape(1, -1))


def _kinit_body(x_ref, e1w, e1b, e2w, e2b, m0w, m0b, b3_ref, br_ref,
                r1w, r1b, r2w, r2b, msrc_ref, r_ref):
    h = _silu(jnp.dot(x_ref[...], e1w[...],
                      preferred_element_type=jnp.float32) + e1b[...])
    h = jnp.dot(h, e2w[...], preferred_element_type=jnp.float32) + e2b[...]
    m0 = jnp.dot(h, m0w[...], preferred_element_type=jnp.float32) + m0b[...]
    for a in range(NORI):
        msrc_ref[:, a, :] = m0
    z = (2.0 * np.pi) * (b3_ref[...] * br_ref[...])
    rff = jnp.concatenate([jnp.cos(z), jnp.sin(z)], axis=1)
    r = _silu(jnp.dot(rff, r1w[...],
                      preferred_element_type=jnp.float32) + r1b[...])
    r = _silu(jnp.dot(r, r2w[...],
                      preferred_element_type=jnp.float32) + r2b[...])
    r_ref[...] = r


def _run_kinit(x, p):
    l0 = p["layers"][0]
    return pl.pallas_call(
        _kinit_body,
        out_shape=[jax.ShapeDtypeStruct((N, NORI, HID), jnp.float32),
                   jax.ShapeDtypeStruct((NORI * NORI, HID), jnp.float32)],
    )(x, p["emb1"]["W"], p["emb1"]["b"].reshape(1, -1),
      p["emb2"]["W"], p["emb2"]["b"].reshape(1, -1),
      l0["msg"]["W"], l0["msg"]["b"].reshape(1, -1),
      jnp.asarray(_B3), p["B_R"],
      p["eR1"]["W"], p["eR1"]["b"].reshape(1, -1),
      p["eR2"]["W"], p["eR2"]["b"].reshape(1, -1))


NC = 2
NS = 16
ROWS_PER_TILE = N // NS
BE = 32


def _sc_mesh():
    return plsc.VectorSubcoreMesh(core_axis_name="c", subcore_axis_name="s",
                                  num_cores=NC, num_subcores=NS)


def _run_compact(posx, posy, posz):
    @functools.partial(
        pl.kernel,
        out_type=[jax.ShapeDtypeStruct((NSEG, 16), jnp.int32),
                  jax.ShapeDtypeStruct((ECAP,), jnp.int32),
                  jax.ShapeDtypeStruct((ECAP,), jnp.int32),
                  jax.ShapeDtypeStruct((ECAP,), jnp.float32),
                  jax.ShapeDtypeStruct((ECAP,), jnp.float32),
                  jax.ShapeDtypeStruct((ECAP,), jnp.float32)],
        mesh=_sc_mesh(),
        scratch_types=[pltpu.VMEM((N + 16,), jnp.float32),
                       pltpu.VMEM((N + 16,), jnp.float32),
                       pltpu.VMEM((N + 16,), jnp.float32),
                       pltpu.VMEM((SEG_CAP,), jnp.int32),
                       pltpu.VMEM((SEG_CAP,), jnp.int32),
                       pltpu.VMEM((SEG_CAP,), jnp.float32),
                       pltpu.VMEM((SEG_CAP,), jnp.float32),
                       pltpu.VMEM((SEG_CAP,), jnp.float32),
                       pltpu.VMEM((16,), jnp.int32)],
    )
    def k(px_h, py_h, pz_h, cnt_h, src_h, dst_h, dx_h, dy_h, dz_h,
          px_v, py_v, pz_v, src_b, dst_b, dxb, dyb, dzb, cnt_v):
        wid = lax.axis_index("s") * NC + lax.axis_index("c")
        pltpu.sync_copy(px_h, px_v.at[pl.ds(0, N)])
        pltpu.sync_copy(py_h, py_v.at[pl.ds(0, N)])
        pltpu.sync_copy(pz_h, pz_v.at[pl.ds(0, N)])
        z16i = jnp.zeros((16,), jnp.int32)
        z16f = jnp.zeros((16,), jnp.float32)

        def zf(i, carry):
            sl = pl.ds(i * 16, 16)
            src_b[sl] = z16i
            dst_b[sl] = z16i
            dxb[sl] = z16f
            dyb[sl] = z16f
            dzb[sl] = z16f
            return carry

        lax.fori_loop(0, SEG_CAP // 16, zf, 0)

        def outer(row, off):
            sg = wid * ROWS_PER_SEG + row
            sx = jnp.full((16,), px_v[pl.ds(sg, 16)][0])
            sy = jnp.full((16,), py_v[pl.ds(sg, 16)][0])
            sz = jnp.full((16,), pz_v[pl.ds(sg, 16)][0])

            def inner(ch, off):
                base = ch * 16
                dxv = sx - px_v[pl.ds(base, 16)]
                dyv = sy - py_v[pl.ds(base, 16)]
                dzv = sz - pz_v[pl.ds(base, 16)]
                d2 = dxv * dxv + dyv * dyv + dzv * dzv
                mi = jnp.where(d2 <= RADIUS * RADIUS, 1, 0)
                for j in range(16):
                    mj = mi[j]
                    ofu = jnp.minimum(off, SEG_CAP - 16)

                    @pl.when(mj == 1)
                    def _(j=j, ofu=ofu, dxv=dxv, dyv=dyv, dzv=dzv, base=base):
                        sl = pl.ds(ofu, 16)
                        dst_b[sl] = jnp.full((16,), base + j, jnp.int32)
                        src_b[sl] = jnp.full((16,), sg, jnp.int32)
                        dxb[sl] = jnp.full((16,), dxv[j])
                        dyb[sl] = jnp.full((16,), dyv[j])
                        dzb[sl] = jnp.full((16,), dzv[j])

                    off = off + mj
                return off

            return lax.fori_loop(0, N // 16, inner, off)

        off = lax.fori_loop(0, ROWS_PER_SEG, outer, jnp.int32(0))
        off = jnp.minimum(off, SEG_CAP)
        base = wid * SEG_CAP
        pltpu.sync_copy(src_b, src_h.at[pl.ds(base, SEG_CAP)])
        pltpu.sync_copy(dst_b, dst_h.at[pl.ds(base, SEG_CAP)])
        pltpu.sync_copy(dxb, dx_h.at[pl.ds(base, SEG_CAP)])
        pltpu.sync_copy(dyb, dy_h.at[pl.ds(base, SEG_CAP)])
        pltpu.sync_copy(dzb, dz_h.at[pl.ds(base, SEG_CAP)])
        cnt_v[...] = jnp.full((16,), off, jnp.int32)
        pltpu.sync_copy(cnt_v, cnt_h.at[wid])

    return k(posx, posy, posz)


def _run_msg_sc(counts2d, src, dst, xt2, msrc2):
    F = NORI * HID
    NR = F // 128
    RPT = (N * NR) // NS

    @functools.partial(
        pl.kernel,
        out_type=jax.ShapeDtypeStruct((NC, N * NR, 128), jnp.float32),
        mesh=_sc_mesh(),
        scratch_types=[pltpu.VMEM((16,), jnp.int32),
                       pltpu.VMEM((BE,), jnp.int32),
                       pltpu.VMEM((BE,), jnp.int32),
                       pltpu.VMEM((NR * BE,), jnp.int32),
                       pltpu.VMEM((BE, F), jnp.float32),
                       pltpu.VMEM((BE, F), jnp.float32),
                       pltpu.VMEM((NR * BE, 128), jnp.float32),
                       pltpu.VMEM((RPT, 128), jnp.float32),
                       pltpu.VMEM_SHARED((N * NR, 128), jnp.float32),
                       pltpu.SemaphoreType.DMA],
    )
    def k(cnt_h, src_h, dst_h, xt_h, ms_h, out_h,
          cnt_v, sidx, didx, idx3, xbuf, mbuf, cbuf, zbuf, agg_sh, sem):
        cid = lax.axis_index("c")
        sid = lax.axis_index("s")
        wid = sid * NC + cid

        def zf2(i, carry):
            r = i // 8
            c2 = (i % 8) * 16
            zbuf[r, pl.ds(c2, 16)] = jnp.zeros((16,), jnp.float32)
            return carry

        lax.fori_loop(0, RPT * 8, zf2, 0)
        pltpu.sync_copy(zbuf, agg_sh.at[pl.ds(sid * RPT, RPT)])
        plsc.subcore_barrier()

        pltpu.sync_copy(cnt_h.at[wid], cnt_v)
        cnt = cnt_v[...][0]
        nblk = (cnt + BE - 1) // BE

        def blk(b, carry):
            base = wid * SEG_CAP + b * BE
            pltpu.sync_copy(src_h.at[pl.ds(base, BE)], sidx)
            pltpu.sync_copy(dst_h.at[pl.ds(base, BE)], didx)
            pltpu.sync_copy(xt_h.at[pl.ds(base, BE)], xbuf)
            pltpu.async_copy(ms_h.at[sidx], mbuf, sem).wait()
            dvec = didx[...]
            for kk in range(NR):
                idx3[pl.ds(kk * BE, BE)] = dvec * NR + kk

            def mul(e, carry2):
                for cc in range(F // 16):
                    kk = cc // 8
                    c2 = (cc % 8) * 16
                    cbuf[kk * BE + e, pl.ds(c2, 16)] = (
                        xbuf[e, pl.ds(cc * 16, 16)]
                        * mbuf[e, pl.ds(cc * 16, 16)])
                return carry2

            lax.fori_loop(0, BE, mul, 0)
            pltpu.sync_copy(cbuf, agg_sh.at[idx3], add=True)
            return carry

        lax.fori_loop(0, nblk, blk, 0)
        plsc.subcore_barrier()
        pltpu.sync_copy(agg_sh.at[pl.ds(sid * RPT, RPT)],
                        out_h.at[cid, pl.ds(sid * RPT, RPT)])

    return k(counts2d, src, dst, xt2, msrc2)


def _make_kupd(i):
    last = (i == 2)
    first = (i == 0)

    def body(*refs):
        it = iter(refs)
        agg_ref = next(it)
        h_prev_ref = None if first else next(it)
        pred_ref = next(it)
        r_ref = None if last else next(it)
        u1w, u1b, u2w, u2b = next(it), next(it), next(it), next(it)
        p1w, p1b, p2w, p2b = next(it), next(it), next(it), next(it)
        if not last:
            mw_, mb_ = next(it), next(it)
        outs = list(it)

        agg = jnp.sum(agg_ref[...], axis=0)
        t = _silu(jnp.dot(agg, u1w[...],
                          preferred_element_type=jnp.float32) + u1b[...])
        upd = jnp.dot(t, u2w[...], preferred_element_type=jnp.float32) + u2b[...]
        h = upd if first else h_prev_ref[...] + upd
        h3 = h.reshape(N, NORI, HID)
        hrd = jnp.sum(h3, axis=1) * (1.0 / NORI)
        tp = _silu(jnp.dot(hrd, p1w[...],
                           preferred_element_type=jnp.float32) + p1b[...])
        pred = pred_ref[...] + jnp.dot(tp, p2w[...],
                                       preferred_element_type=jnp.float32) + p2b[...]
        if last:
            outs[0][...] = jnp.sum(pred, axis=0, keepdims=True)
        else:
            h_out, msrc_out, pred_out = outs
            h_out[...] = h
            pred_out[...] = pred
            hm = jnp.dot(h, mw_[...],
                         preferred_element_type=jnp.float32) + mb_[...]
            hm3 = hm.reshape(N, NORI, HID)
            for b in range(NORI):
                acc = hm3[:, 0, :] * r_ref[0 * NORI + b, :][None, :]
                for a in range(1, NORI):
                    acc = acc + hm3[:, a, :] * r_ref[a * NORI + b, :][None, :]
                msrc_out[:, b, :] = acc

    return body


def _run_kupd(i, agg, h_prev, pred, r, p):
    last = (i == 2)
    lp = p["layers"][i]
    pp = p["post"][i]
    args = [agg]
    if i != 0:
        args.append(h_prev)
    args.append(pred)
    if not last:
        args.append(r)
    args += [lp["u1"]["W"], lp["u1"]["b"].reshape(1, -1),
             lp["u2"]["W"], lp["u2"]["b"].reshape(1, -1),
             pp["p1"]["W"], pp["p1"]["b"].reshape(1, -1),
             pp["p2"]["W"], pp["p2"]["b"].reshape(1, -1)]
    if not last:
        ln = p["layers"][i + 1]
        args += [ln["msg"]["W"], ln["msg"]["b"].reshape(1, -1)]
    if last:
        out_shape = [jax.ShapeDtypeStruct((1, OUT), jnp.float32)]
    else:
        out_shape = [jax.ShapeDtypeStruct((N * NORI, HID), jnp.float32),
                     jax.ShapeDtypeStruct((N, NORI, HID), jnp.float32),
                     jax.ShapeDtypeStruct((N, OUT), jnp.float32)]
    return pl.pallas_call(_make_kupd(i), out_shape=out_shape)(*args)


def kernel(pos, x, batch, params):
    px = pos[:, 0]
    py = pos[:, 1]
    pz = pos[:, 2]
    counts2d, src, dst, dxe, dye, dze = _run_compact(px, py, pz)
    counts = counts2d[:, 0]
    xt = _run_kx(counts, dxe.reshape(ECAP, 1), dye.reshape(ECAP, 1),
                 dze.reshape(ECAP, 1), params)
    xt2 = xt.reshape(ECAP, NORI * HID)
    msrc, r = _run_kinit(x, params)
    h = None
    pred = jnp.zeros((N, OUT), jnp.float32)
    for i in range(3):
        aggp = _run_msg_sc(counts2d, src, dst, xt2, msrc.reshape(N, NORI * HID))
        agg = aggp.reshape(NC, N * NORI, HID)
        if i < 2:
            h, msrc, pred = _run_kupd(i, agg, h, pred, r, params)
        else:
            (out,) = _run_kupd(i, agg, h, pred, None, params)
    return out


# speedup vs baseline: 4.1037x; 1.1323x over previous
import functools
import numpy as np
import jax
import jax.numpy as jnp
from jax import lax
from jax.experimental import pallas as pl
from jax.experimental.pallas import tpu as pltpu
from jax.experimental.pallas import tpu_sc as plsc

N = 1024
NORI = 6
HID = 64
OUT = 32
NSEG = 32
ROWS_PER_SEG = N // NSEG
SEG_CAP = 2048
ECAP = NSEG * SEG_CAP
MSG_BLK = 512
RADIUS = 2.0


def _fib_sphere(n):
    i = np.arange(n, dtype=np.float64) + 0.5
    phi = np.arccos(1.0 - 2.0 * i / n)
    theta = np.pi * (1.0 + 5.0 ** 0.5) * i
    return np.stack([np.cos(theta) * np.sin(phi), np.sin(theta) * np.sin(phi),
                     np.cos(phi)], axis=-1).astype(np.float32)

_ORI = _fib_sphere(NORI)
_B3 = np.arccos(np.clip((_ORI @ _ORI.T), -1.0 + 1e-6, 1.0 - 1e-6))
_B3 = _B3.reshape(NORI * NORI, 1).astype(np.float32)


def _silu(v):
    return v / (1.0 + jnp.exp(-v))


def _fit_sincos():
    th = np.linspace(-np.pi, np.pi, 8001)
    t = th * th
    cc = np.polyfit(t, np.cos(th), 6)[::-1]
    ss = np.polyfit(t[t > 1e-12], np.sin(th[t > 1e-12]) / th[t > 1e-12], 6)[::-1]
    return cc.astype(np.float64), ss.astype(np.float64)

_CC, _SS = _fit_sincos()


def _sincos_2pi(u):
    f = u - jnp.round(u)
    th = (2.0 * np.pi) * f
    t = th * th
    c = jnp.float32(_CC[6])
    s = jnp.float32(_SS[6])
    for k in range(5, -1, -1):
        c = c * t + jnp.float32(_CC[k])
        s = s * t + jnp.float32(_SS[k])
    return c, th * s


XBLK = 256
XSUB = SEG_CAP // XBLK


def _kx_body(counts_ref, dx_ref, dy_ref, dz_ref, bx_ref, w1_ref, b1_ref,
             w2_ref, b2_ref, out_ref):
    i = pl.program_id(0)
    seg = i // XSUB
    base = (i % XSUB) * XBLK
    cnt = counts_ref[seg]

    @pl.when(base < cnt)
    def _():
        dx = dx_ref[...]
        dy = dy_ref[...]
        dz = dz_ref[...]
        d2 = dx * dx + dy * dy + dz * dz
        dist = jnp.sqrt(d2 + 1e-12)
        mw = 0.5 * (jnp.cos((np.pi / RADIUS) * dist) + 1.0)
        eidx = jax.lax.broadcasted_iota(jnp.int32, (XBLK, 1), 0) + base
        valid = eidx < cnt
        wt = jnp.where(valid, mw, 0.0)
        b0 = bx_ref[0:1, :]
        b1r = bx_ref[1:2, :]
        for a in range(NORI):
            ox, oy, oz = (float(_ORI[a, 0]), float(_ORI[a, 1]),
                          float(_ORI[a, 2]))
            a1 = dx * ox + dy * oy + dz * oz
            px = dx - a1 * ox
            py = dy - a1 * oy
            pz = dz - a1 * oz
            a2 = jnp.sqrt(px * px + py * py + pz * pz + 1e-12)
            u = a1 * b0 + a2 * b1r
            cz, sz = _sincos_2pi(u)
            rff = jnp.concatenate([cz, sz], axis=1)
            h = _silu(jnp.dot(rff, w1_ref[...],
                              preferred_element_type=jnp.float32) + b1_ref[...])
            h = _silu(jnp.dot(h, w2_ref[...],
                              preferred_element_type=jnp.float32) + b2_ref[...])
            out_ref[:, a, :] = jnp.where(valid, h * wt, 0.0)


def _run_kx(counts, dxc, dyc, dzc, p):
    full = lambda shape: pl.BlockSpec(shape, lambda i, c: tuple(0 for _ in shape))
    return pl.pallas_call(
        _kx_body,
        grid_spec=pltpu.PrefetchScalarGridSpec(
            num_scalar_prefetch=1,
            grid=(NSEG * XSUB,),
            in_specs=[
                pl.BlockSpec((XBLK, 1), lambda i, c: (i, 0)),
                pl.BlockSpec((XBLK, 1), lambda i, c: (i, 0)),
                pl.BlockSpec((XBLK, 1), lambda i, c: (i, 0)),
                full((2, HID // 2)),
                full((HID, HID)), full((1, HID)),
                full((HID, HID)), full((1, HID)),
            ],
            out_specs=pl.BlockSpec((XBLK, NORI, HID), lambda i, c: (i, 0, 0)),
        ),
        out_shape=jax.ShapeDtypeStruct((ECAP, NORI, HID), jnp.float32),
    )(counts, dxc, dyc, dzc, p["B_x"],
      p["ex1"]["W"], p["ex1"]["b"].reshape(1, -1),
      p["ex2"]["W"], p["ex2"]["b"].reshape(1, -1))


def _kinit_body(x_ref, e1w, e1b, e2w, e2b, m0w, m0b, b3_ref, br_ref,
                r1w, r1b, r2w, r2b, msrc_ref, r_ref):
    h = _silu(jnp.dot(x_ref[...], e1w[...],
                      preferred_element_type=jnp.float32) + e1b[...])
    h = jnp.dot(h, e2w[...], preferred_element_type=jnp.float32) + e2b[...]
    m0 = jnp.dot(h, m0w[...], preferred_element_type=jnp.float32) + m0b[...]
    for a in range(NORI):
        msrc_ref[:, a, :] = m0
    z = (2.0 * np.pi) * (b3_ref[...] * br_ref[...])
    rff = jnp.concatenate([jnp.cos(z), jnp.sin(z)], axis=1)
    r = _silu(jnp.dot(rff, r1w[...],
                      preferred_element_type=jnp.float32) + r1b[...])
    r = _silu(jnp.dot(r, r2w[...],
                      preferred_element_type=jnp.float32) + r2b[...])
    r_ref[...] = r


def _run_kinit(x, p):
    l0 = p["layers"][0]
    return pl.pallas_call(
        _kinit_body,
        out_shape=[jax.ShapeDtypeStruct((N, NORI, HID), jnp.float32),
                   jax.ShapeDtypeStruct((NORI * NORI, HID), jnp.float32)],
    )(x, p["emb1"]["W"], p["emb1"]["b"].reshape(1, -1),
      p["emb2"]["W"], p["emb2"]["b"].reshape(1, -1),
      l0["msg"]["W"], l0["msg"]["b"].reshape(1, -1),
      jnp.asarray(_B3), p["B_R"],
      p["eR1"]["W"], p["eR1"]["b"].reshape(1, -1),
      p["eR2"]["W"], p["eR2"]["b"].reshape(1, -1))


NC = 2
NS = 16
ROWS_PER_TILE = N // NS
BE = 32


def _sc_mesh():
    return plsc.VectorSubcoreMesh(core_axis_name="c", subcore_axis_name="s",
                                  num_cores=NC, num_subcores=NS)


def _run_compact(posx, posy, posz):
    @functools.partial(
        pl.kernel,
        out_type=[jax.ShapeDtypeStruct((NSEG, 16), jnp.int32),
                  jax.ShapeDtypeStruct((ECAP,), jnp.int32),
                  jax.ShapeDtypeStruct((ECAP,), jnp.int32),
                  jax.ShapeDtypeStruct((ECAP,), jnp.float32),
                  jax.ShapeDtypeStruct((ECAP,), jnp.float32),
                  jax.ShapeDtypeStruct((ECAP,), jnp.float32)],
        mesh=_sc_mesh(),
        scratch_types=[pltpu.VMEM((N + 16,), jnp.float32),
                       pltpu.VMEM((N + 16,), jnp.float32),
                       pltpu.VMEM((N + 16,), jnp.float32),
                       pltpu.VMEM((SEG_CAP,), jnp.int32),
                       pltpu.VMEM((SEG_CAP,), jnp.int32),
                       pltpu.VMEM((SEG_CAP,), jnp.float32),
                       pltpu.VMEM((SEG_CAP,), jnp.float32),
                       pltpu.VMEM((SEG_CAP,), jnp.float32),
                       pltpu.VMEM((16,), jnp.int32)],
    )
    def k(px_h, py_h, pz_h, cnt_h, src_h, dst_h, dx_h, dy_h, dz_h,
          px_v, py_v, pz_v, src_b, dst_b, dxb, dyb, dzb, cnt_v):
        wid = lax.axis_index("s") * NC + lax.axis_index("c")
        pltpu.sync_copy(px_h, px_v.at[pl.ds(0, N)])
        pltpu.sync_copy(py_h, py_v.at[pl.ds(0, N)])
        pltpu.sync_copy(pz_h, pz_v.at[pl.ds(0, N)])
        z16i = jnp.zeros((16,), jnp.int32)
        z16f = jnp.zeros((16,), jnp.float32)

        def zf(i, carry):
            sl = pl.ds(i * 16, 16)
            src_b[sl] = z16i
            dst_b[sl] = z16i
            dxb[sl] = z16f
            dyb[sl] = z16f
            dzb[sl] = z16f
            return carry

        lax.fori_loop(0, SEG_CAP // 16, zf, 0)

        def outer(row, off):
            sg = wid * ROWS_PER_SEG + row
            sx = jnp.full((16,), px_v[pl.ds(sg, 16)][0])
            sy = jnp.full((16,), py_v[pl.ds(sg, 16)][0])
            sz = jnp.full((16,), pz_v[pl.ds(sg, 16)][0])

            def inner(ch, off):
                base = ch * 16
                dxv = sx - px_v[pl.ds(base, 16)]
                dyv = sy - py_v[pl.ds(base, 16)]
                dzv = sz - pz_v[pl.ds(base, 16)]
                d2 = dxv * dxv + dyv * dyv + dzv * dzv
                mi = jnp.where(d2 <= RADIUS * RADIUS, 1, 0)
                for j in range(16):
                    mj = mi[j]
                    ofu = jnp.minimum(off, SEG_CAP - 16)

                    @pl.when(mj == 1)
                    def _(j=j, ofu=ofu, dxv=dxv, dyv=dyv, dzv=dzv, base=base):
                        sl = pl.ds(ofu, 16)
                        dst_b[sl] = jnp.full((16,), base + j, jnp.int32)
                        src_b[sl] = jnp.full((16,), sg, jnp.int32)
                        dxb[sl] = jnp.full((16,), dxv[j])
                        dyb[sl] = jnp.full((16,), dyv[j])
                        dzb[sl] = jnp.full((16,), dzv[j])

                    off = off + mj
                return off

            return lax.fori_loop(0, N // 16, inner, off)

        off = lax.fori_loop(0, ROWS_PER_SEG, outer, jnp.int32(0))
        off = jnp.minimum(off, SEG_CAP)
        base = wid * SEG_CAP
        pltpu.sync_copy(src_b, src_h.at[pl.ds(base, SEG_CAP)])
        pltpu.sync_copy(dst_b, dst_h.at[pl.ds(base, SEG_CAP)])
        pltpu.sync_copy(dxb, dx_h.at[pl.ds(base, SEG_CAP)])
        pltpu.sync_copy(dyb, dy_h.at[pl.ds(base, SEG_CAP)])
        pltpu.sync_copy(dzb, dz_h.at[pl.ds(base, SEG_CAP)])
        cnt_v[...] = jnp.full((16,), off, jnp.int32)
        pltpu.sync_copy(cnt_v, cnt_h.at[wid])

    return k(posx, posy, posz)


def _run_msg_sc(counts2d, src, dst, xt2, msrc2):
    F = NORI * HID
    NR = F // 128
    RPT = (N * NR) // NS

    @functools.partial(
        pl.kernel,
        out_type=jax.ShapeDtypeStruct((NC, N * NR, 128), jnp.float32),
        mesh=_sc_mesh(),
        scratch_types=[pltpu.VMEM((16,), jnp.int32),
                       pltpu.VMEM((BE,), jnp.int32),
                       pltpu.VMEM((BE,), jnp.int32),
                       pltpu.VMEM((NR * BE,), jnp.int32),
                       pltpu.VMEM((BE, F), jnp.float32),
                       pltpu.VMEM((BE, F), jnp.float32),
                       pltpu.VMEM((NR * BE, 128), jnp.float32),
                       pltpu.VMEM((RPT, 128), jnp.float32),
                       pltpu.VMEM_SHARED((N * NR, 128), jnp.float32),
                       pltpu.SemaphoreType.DMA],
    )
    def k(cnt_h, src_h, dst_h, xt_h, ms_h, out_h,
          cnt_v, sidx, didx, idx3, xbuf, mbuf, cbuf, zbuf, agg_sh, sem):
        cid = lax.axis_index("c")
        sid = lax.axis_index("s")
        wid = sid * NC + cid

        def zf2(i, carry):
            r = i // 8
            c2 = (i % 8) * 16
            zbuf[r, pl.ds(c2, 16)] = jnp.zeros((16,), jnp.float32)
            return carry

        lax.fori_loop(0, RPT * 8, zf2, 0)
        pltpu.sync_copy(zbuf, agg_sh.at[pl.ds(sid * RPT, RPT)])
        plsc.subcore_barrier()

        pltpu.sync_copy(cnt_h.at[wid], cnt_v)
        cnt = cnt_v[...][0]
        nblk = (cnt + BE - 1) // BE

        def blk(b, carry):
            base = wid * SEG_CAP + b * BE
            pltpu.sync_copy(src_h.at[pl.ds(base, BE)], sidx)
            pltpu.sync_copy(dst_h.at[pl.ds(base, BE)], didx)
            pltpu.sync_copy(xt_h.at[pl.ds(base, BE)], xbuf)
            pltpu.async_copy(ms_h.at[sidx], mbuf, sem).wait()
            dvec = didx[...]
            for kk in range(NR):
                idx3[pl.ds(kk * BE, BE)] = dvec * NR + kk

            def mul(e, carry2):
                for cc in range(F // 16):
                    kk = cc // 8
                    c2 = (cc % 8) * 16
                    cbuf[kk * BE + e, pl.ds(c2, 16)] = (
                        xbuf[e, pl.ds(cc * 16, 16)]
                        * mbuf[e, pl.ds(cc * 16, 16)])
                return carry2

            lax.fori_loop(0, BE, mul, 0)
            pltpu.sync_copy(cbuf, agg_sh.at[idx3], add=True)
            return carry

        lax.fori_loop(0, nblk, blk, 0)
        plsc.subcore_barrier()
        pltpu.sync_copy(agg_sh.at[pl.ds(sid * RPT, RPT)],
                        out_h.at[cid, pl.ds(sid * RPT, RPT)])

    return k(counts2d, src, dst, xt2, msrc2)


def _make_kupd(i):
    last = (i == 2)
    first = (i == 0)

    def body(*refs):
        it = iter(refs)
        agg_ref = next(it)
        h_prev_ref = None if first else next(it)
        pred_ref = next(it)
        r_ref = None if last else next(it)
        u1w, u1b, u2w, u2b = next(it), next(it), next(it), next(it)
        p1w, p1b, p2w, p2b = next(it), next(it), next(it), next(it)
        if not last:
            mw_, mb_ = next(it), next(it)
        outs = list(it)

        agg = jnp.sum(agg_ref[...], axis=0)
        t = _silu(jnp.dot(agg, u1w[...],
                          preferred_element_type=jnp.float32) + u1b[...])
        upd = jnp.dot(t, u2w[...], preferred_element_type=jnp.float32) + u2b[...]
        h = upd if first else h_prev_ref[...] + upd
        h3 = h.reshape(N, NORI, HID)
        hrd = jnp.sum(h3, axis=1) * (1.0 / NORI)
        tp = _silu(jnp.dot(hrd, p1w[...],
                           preferred_element_type=jnp.float32) + p1b[...])
        pred = pred_ref[...] + jnp.dot(tp, p2w[...],
                                       preferred_element_type=jnp.float32) + p2b[...]
        if last:
            outs[0][...] = jnp.sum(pred, axis=0, keepdims=True)
        else:
            h_out, msrc_out, pred_out = outs
            h_out[...] = h
            pred_out[...] = pred
            hm = jnp.dot(h, mw_[...],
                         preferred_element_type=jnp.float32) + mb_[...]
            hm3 = hm.reshape(N, NORI, HID)
            for b in range(NORI):
                acc = hm3[:, 0, :] * r_ref[0 * NORI + b, :][None, :]
                for a in range(1, NORI):
                    acc = acc + hm3[:, a, :] * r_ref[a * NORI + b, :][None, :]
                msrc_out[:, b, :] = acc

    return body


def _run_kupd(i, agg, h_prev, pred, r, p):
    last = (i == 2)
    lp = p["layers"][i]
    pp = p["post"][i]
    args = [agg]
    if i != 0:
        args.append(h_prev)
    args.append(pred)
    if not last:
        args.append(r)
    args += [lp["u1"]["W"], lp["u1"]["b"].reshape(1, -1),
             lp["u2"]["W"], lp["u2"]["b"].reshape(1, -1),
             pp["p1"]["W"], pp["p1"]["b"].reshape(1, -1),
             pp["p2"]["W"], pp["p2"]["b"].reshape(1, -1)]
    if not last:
        ln = p["layers"][i + 1]
        args += [ln["msg"]["W"], ln["msg"]["b"].reshape(1, -1)]
    if last:
        out_shape = [jax.ShapeDtypeStruct((1, OUT), jnp.float32)]
    else:
        out_shape = [jax.ShapeDtypeStruct((N * NORI, HID), jnp.float32),
                     jax.ShapeDtypeStruct((N, NORI, HID), jnp.float32),
                     jax.ShapeDtypeStruct((N, OUT), jnp.float32)]
    return pl.pallas_call(_make_kupd(i), out_shape=out_shape)(*args)


def kernel(pos, x, batch, params):
    px = pos[:, 0]
    py = pos[:, 1]
    pz = pos[:, 2]
    counts2d, src, dst, dxe, dye, dze = _run_compact(px, py, pz)
    counts = counts2d[:, 0]
    xt = _run_kx(counts, dxe.reshape(ECAP, 1), dye.reshape(ECAP, 1),
                 dze.reshape(ECAP, 1), params)
    xt2 = xt.reshape(ECAP, NORI * HID)
    msrc, r = _run_kinit(x, params)
    h = None
    pred = jnp.zeros((N, OUT), jnp.float32)
    for i in range(3):
        aggp = _run_msg_sc(counts2d, src, dst, xt2, msrc.reshape(N, NORI * HID))
        agg = aggp.reshape(NC, N * NORI, HID)
        if i < 2:
            h, msrc, pred = _run_kupd(i, agg, h, pred, r, params)
        else:
            (out,) = _run_kupd(i, agg, h, pred, None, params)
    return out


# speedup vs baseline: 4.8796x; 1.1891x over previous
import functools
import numpy as np
import jax
import jax.numpy as jnp
from jax import lax
from jax.experimental import pallas as pl
from jax.experimental.pallas import tpu as pltpu
from jax.experimental.pallas import tpu_sc as plsc

N = 1024
NORI = 6
HID = 64
OUT = 32
NSEG = 32
ROWS_PER_SEG = N // NSEG
SEG_CAP = 2048
ECAP = NSEG * SEG_CAP
MSG_BLK = 512
RADIUS = 2.0


def _fib_sphere(n):
    i = np.arange(n, dtype=np.float64) + 0.5
    phi = np.arccos(1.0 - 2.0 * i / n)
    theta = np.pi * (1.0 + 5.0 ** 0.5) * i
    return np.stack([np.cos(theta) * np.sin(phi), np.sin(theta) * np.sin(phi),
                     np.cos(phi)], axis=-1).astype(np.float32)

_ORI = _fib_sphere(NORI)
_B3 = np.arccos(np.clip((_ORI @ _ORI.T), -1.0 + 1e-6, 1.0 - 1e-6))
_B3 = _B3.reshape(NORI * NORI, 1).astype(np.float32)


def _silu(v):
    return v / (1.0 + jnp.exp(-v))


def _fit_sincos():
    th = np.linspace(-np.pi, np.pi, 8001)
    t = th * th
    cc = np.polyfit(t, np.cos(th), 6)[::-1]
    ss = np.polyfit(t[t > 1e-12], np.sin(th[t > 1e-12]) / th[t > 1e-12], 6)[::-1]
    return cc.astype(np.float64), ss.astype(np.float64)

_CC, _SS = _fit_sincos()


def _sincos_2pi(u):
    f = u - jnp.round(u)
    th = (2.0 * np.pi) * f
    t = th * th
    c = jnp.float32(_CC[6])
    s = jnp.float32(_SS[6])
    for k in range(5, -1, -1):
        c = c * t + jnp.float32(_CC[k])
        s = s * t + jnp.float32(_SS[k])
    return c, th * s


XBLK = 256
XSUB = SEG_CAP // XBLK


def _kx_body(counts_ref, dx_ref, dy_ref, dz_ref, bx_ref, w1_ref, b1_ref,
             w2_ref, b2_ref, out_ref):
    i = pl.program_id(0)
    seg = i // XSUB
    base = (i % XSUB) * XBLK
    cnt = counts_ref[seg]

    @pl.when(base < cnt)
    def _():
        dx = dx_ref[...]
        dy = dy_ref[...]
        dz = dz_ref[...]
        d2 = dx * dx + dy * dy + dz * dz
        dist = jnp.sqrt(d2 + 1e-12)
        mw = 0.5 * (jnp.cos((np.pi / RADIUS) * dist) + 1.0)
        eidx = jax.lax.broadcasted_iota(jnp.int32, (XBLK, 1), 0) + base
        valid = eidx < cnt
        wt = jnp.where(valid, mw, 0.0)
        b0 = bx_ref[0:1, :]
        b1r = bx_ref[1:2, :]
        for a in range(NORI):
            ox, oy, oz = (float(_ORI[a, 0]), float(_ORI[a, 1]),
                          float(_ORI[a, 2]))
            a1 = dx * ox + dy * oy + dz * oz
            px = dx - a1 * ox
            py = dy - a1 * oy
            pz = dz - a1 * oz
            a2 = jnp.sqrt(px * px + py * py + pz * pz + 1e-12)
            u = a1 * b0 + a2 * b1r
            cz, sz = _sincos_2pi(u)
            rff = jnp.concatenate([cz, sz], axis=1)
            h = _silu(jnp.dot(rff, w1_ref[...],
                              preferred_element_type=jnp.float32) + b1_ref[...])
            h = _silu(jnp.dot(h, w2_ref[...],
                              preferred_element_type=jnp.float32) + b2_ref[...])
            out_ref[:, a, :] = jnp.where(valid, h * wt, 0.0)


def _run_kx(counts, dxc, dyc, dzc, p):
    full = lambda shape: pl.BlockSpec(shape, lambda i, c: tuple(0 for _ in shape))
    return pl.pallas_call(
        _kx_body,
        grid_spec=pltpu.PrefetchScalarGridSpec(
            num_scalar_prefetch=1,
            grid=(NSEG * XSUB,),
            in_specs=[
                pl.BlockSpec((XBLK, 1), lambda i, c: (i, 0)),
                pl.BlockSpec((XBLK, 1), lambda i, c: (i, 0)),
                pl.BlockSpec((XBLK, 1), lambda i, c: (i, 0)),
                full((2, HID // 2)),
                full((HID, HID)), full((1, HID)),
                full((HID, HID)), full((1, HID)),
            ],
            out_specs=pl.BlockSpec((XBLK, NORI, HID), lambda i, c: (i, 0, 0)),
        ),
        out_shape=jax.ShapeDtypeStruct((ECAP, NORI, HID), jnp.float32),
    )(counts, dxc, dyc, dzc, p["B_x"],
      p["ex1"]["W"], p["ex1"]["b"].reshape(1, -1),
      p["ex2"]["W"], p["ex2"]["b"].reshape(1, -1))


def _kinit_body(x_ref, e1w, e1b, e2w, e2b, m0w, m0b, b3_ref, br_ref,
                r1w, r1b, r2w, r2b, msrc_ref, r_ref):
    h = _silu(jnp.dot(x_ref[...], e1w[...],
                      preferred_element_type=jnp.float32) + e1b[...])
    h = jnp.dot(h, e2w[...], preferred_element_type=jnp.float32) + e2b[...]
    m0 = jnp.dot(h, m0w[...], preferred_element_type=jnp.float32) + m0b[...]
    for a in range(NORI):
        msrc_ref[:, a, :] = m0
    z = (2.0 * np.pi) * (b3_ref[...] * br_ref[...])
    rff = jnp.concatenate([jnp.cos(z), jnp.sin(z)], axis=1)
    r = _silu(jnp.dot(rff, r1w[...],
                      preferred_element_type=jnp.float32) + r1b[...])
    r = _silu(jnp.dot(r, r2w[...],
                      preferred_element_type=jnp.float32) + r2b[...])
    r_ref[...] = r


def _run_kinit(x, p):
    l0 = p["layers"][0]
    return pl.pallas_call(
        _kinit_body,
        out_shape=[jax.ShapeDtypeStruct((N, NORI, HID), jnp.float32),
                   jax.ShapeDtypeStruct((NORI * NORI, HID), jnp.float32)],
    )(x, p["emb1"]["W"], p["emb1"]["b"].reshape(1, -1),
      p["emb2"]["W"], p["emb2"]["b"].reshape(1, -1),
      l0["msg"]["W"], l0["msg"]["b"].reshape(1, -1),
      jnp.asarray(_B3), p["B_R"],
      p["eR1"]["W"], p["eR1"]["b"].reshape(1, -1),
      p["eR2"]["W"], p["eR2"]["b"].reshape(1, -1))


NC = 2
NS = 16
ROWS_PER_TILE = N // NS
BE = 32


def _sc_mesh():
    return plsc.VectorSubcoreMesh(core_axis_name="c", subcore_axis_name="s",
                                  num_cores=NC, num_subcores=NS)


def _run_compact(posx, posy, posz):
    @functools.partial(
        pl.kernel,
        out_type=[jax.ShapeDtypeStruct((NSEG, 16), jnp.int32),
                  jax.ShapeDtypeStruct((ECAP,), jnp.int32),
                  jax.ShapeDtypeStruct((ECAP,), jnp.int32),
                  jax.ShapeDtypeStruct((ECAP,), jnp.float32),
                  jax.ShapeDtypeStruct((ECAP,), jnp.float32),
                  jax.ShapeDtypeStruct((ECAP,), jnp.float32)],
        mesh=_sc_mesh(),
        scratch_types=[pltpu.VMEM((N + 16,), jnp.float32),
                       pltpu.VMEM((N + 16,), jnp.float32),
                       pltpu.VMEM((N + 16,), jnp.float32),
                       pltpu.VMEM((SEG_CAP,), jnp.int32),
                       pltpu.VMEM((SEG_CAP,), jnp.int32),
                       pltpu.VMEM((SEG_CAP,), jnp.float32),
                       pltpu.VMEM((SEG_CAP,), jnp.float32),
                       pltpu.VMEM((SEG_CAP,), jnp.float32),
                       pltpu.VMEM((16,), jnp.int32)],
    )
    def k(px_h, py_h, pz_h, cnt_h, src_h, dst_h, dx_h, dy_h, dz_h,
          px_v, py_v, pz_v, src_b, dst_b, dxb, dyb, dzb, cnt_v):
        wid = lax.axis_index("s") * NC + lax.axis_index("c")
        pltpu.sync_copy(px_h, px_v.at[pl.ds(0, N)])
        pltpu.sync_copy(py_h, py_v.at[pl.ds(0, N)])
        pltpu.sync_copy(pz_h, pz_v.at[pl.ds(0, N)])
        z16i = jnp.zeros((16,), jnp.int32)
        z16f = jnp.zeros((16,), jnp.float32)

        def zf(i, carry):
            sl = pl.ds(i * 16, 16)
            src_b[sl] = z16i
            dst_b[sl] = z16i
            dxb[sl] = z16f
            dyb[sl] = z16f
            dzb[sl] = z16f
            return carry

        lax.fori_loop(0, SEG_CAP // 16, zf, 0)

        def outer(row, off):
            sg = wid * ROWS_PER_SEG + row
            sx = jnp.full((16,), px_v[pl.ds(sg, 16)][0])
            sy = jnp.full((16,), py_v[pl.ds(sg, 16)][0])
            sz = jnp.full((16,), pz_v[pl.ds(sg, 16)][0])

            def inner(ch, off):
                base = ch * 16
                dxv = sx - px_v[pl.ds(base, 16)]
                dyv = sy - py_v[pl.ds(base, 16)]
                dzv = sz - pz_v[pl.ds(base, 16)]
                d2 = dxv * dxv + dyv * dyv + dzv * dzv
                mi = jnp.where(d2 <= RADIUS * RADIUS, 1, 0)
                for j in range(16):
                    mj = mi[j]
                    ofu = jnp.minimum(off, SEG_CAP - 16)

                    @pl.when(mj == 1)
                    def _(j=j, ofu=ofu, dxv=dxv, dyv=dyv, dzv=dzv, base=base):
                        sl = pl.ds(ofu, 16)
                        dst_b[sl] = jnp.full((16,), base + j, jnp.int32)
                        src_b[sl] = jnp.full((16,), sg, jnp.int32)
                        dxb[sl] = jnp.full((16,), dxv[j])
                        dyb[sl] = jnp.full((16,), dyv[j])
                        dzb[sl] = jnp.full((16,), dzv[j])

                    off = off + mj
                return off

            return lax.fori_loop(0, N // 16, inner, off)

        off = lax.fori_loop(0, ROWS_PER_SEG, outer, jnp.int32(0))
        off = jnp.minimum(off, SEG_CAP)
        base = wid * SEG_CAP
        pltpu.sync_copy(src_b, src_h.at[pl.ds(base, SEG_CAP)])
        pltpu.sync_copy(dst_b, dst_h.at[pl.ds(base, SEG_CAP)])
        pltpu.sync_copy(dxb, dx_h.at[pl.ds(base, SEG_CAP)])
        pltpu.sync_copy(dyb, dy_h.at[pl.ds(base, SEG_CAP)])
        pltpu.sync_copy(dzb, dz_h.at[pl.ds(base, SEG_CAP)])
        cnt_v[...] = jnp.full((16,), off, jnp.int32)
        pltpu.sync_copy(cnt_v, cnt_h.at[wid])

    return k(posx, posy, posz)


def _run_msg_sc(counts2d, src, dst, xt2, msrc2):
    F = NORI * HID
    NR = F // 128
    RPT = (N * NR) // NS

    @functools.partial(
        pl.kernel,
        out_type=jax.ShapeDtypeStruct((NC, N * NR, 128), jnp.float32),
        mesh=_sc_mesh(),
        scratch_types=[pltpu.VMEM((16,), jnp.int32),
                       pltpu.VMEM((SEG_CAP,), jnp.int32),
                       pltpu.VMEM((SEG_CAP,), jnp.int32),
                       pltpu.VMEM((NR * BE,), jnp.int32),
                       pltpu.VMEM((BE, F), jnp.float32),
                       pltpu.VMEM((BE, F), jnp.float32),
                       pltpu.VMEM((BE, F), jnp.float32),
                       pltpu.VMEM((BE, F), jnp.float32),
                       pltpu.VMEM((NR * BE, 128), jnp.float32),
                       pltpu.VMEM((RPT, 128), jnp.float32),
                       pltpu.VMEM_SHARED((N * NR, 128), jnp.float32),
                       pltpu.SemaphoreType.DMA,
                       pltpu.SemaphoreType.DMA,
                       pltpu.SemaphoreType.DMA,
                       pltpu.SemaphoreType.DMA],
    )
    def k(cnt_h, src_h, dst_h, xt_h, ms_h, out_h,
          cnt_v, sidx, didx, idx3, xb0, xb1, mb0, mb1, cbuf, zbuf, agg_sh,
          sx0, sx1, sg0, sg1):
        cid = lax.axis_index("c")
        sid = lax.axis_index("s")
        wid = sid * NC + cid
        xb = (xb0, xb1)
        mb = (mb0, mb1)
        sx = (sx0, sx1)
        sg = (sg0, sg1)

        def zf2(i, carry):
            r = i // 8
            c2 = (i % 8) * 16
            zbuf[r, pl.ds(c2, 16)] = jnp.zeros((16,), jnp.float32)
            return carry

        lax.fori_loop(0, RPT * 8, zf2, 0)
        pltpu.sync_copy(zbuf, agg_sh.at[pl.ds(sid * RPT, RPT)])
        plsc.subcore_barrier()

        pltpu.sync_copy(cnt_h.at[wid], cnt_v)
        cnt = cnt_v[...][0]
        nblk = (cnt + BE - 1) // BE
        seg0 = wid * SEG_CAP
        pltpu.sync_copy(src_h.at[pl.ds(seg0, SEG_CAP)], sidx)
        pltpu.sync_copy(dst_h.at[pl.ds(seg0, SEG_CAP)], didx)

        def issue(j, par):
            pltpu.async_copy(xt_h.at[pl.ds(seg0 + j * BE, BE)], xb[par],
                             sx[par])
            pltpu.async_copy(ms_h.at[sidx.at[pl.ds(j * BE, BE)]], mb[par],
                             sg[par])

        issue(0, 0)

        def pair(b2, carry):
            for par in (0, 1):
                j = b2 * 2 + par

                @pl.when(j < nblk)
                def _(j=j, par=par):
                    @pl.when(j + 1 < nblk)
                    def _():
                        issue(j + 1, 1 - par)

                    pltpu.make_async_copy(
                        xt_h.at[pl.ds(seg0 + j * BE, BE)], xb[par],
                        sx[par]).wait()
                    pltpu.make_async_copy(
                        ms_h.at[sidx.at[pl.ds(j * BE, BE)]], mb[par],
                        sg[par]).wait()
                    for kk in range(NR):
                        for h in range(BE // 16):
                            idx3[pl.ds(kk * BE + h * 16, 16)] = (
                                didx[pl.ds(j * BE + h * 16, 16)] * NR + kk)

                    xbuf = xb[par]
                    mbuf = mb[par]

                    def mul(e, carry2):
                        for cc in range(F // 16):
                            kk = cc // 8
                            c2 = (cc % 8) * 16
                            cbuf[kk * BE + e, pl.ds(c2, 16)] = (
                                xbuf[e, pl.ds(cc * 16, 16)]
                                * mbuf[e, pl.ds(cc * 16, 16)])
                        return carry2

                    lax.fori_loop(0, BE, mul, 0)
                    pltpu.sync_copy(cbuf, agg_sh.at[idx3], add=True)

            return carry

        lax.fori_loop(0, (nblk + 1) // 2, pair, 0)
        plsc.subcore_barrier()
        pltpu.sync_copy(agg_sh.at[pl.ds(sid * RPT, RPT)],
                        out_h.at[cid, pl.ds(sid * RPT, RPT)])

    return k(counts2d, src, dst, xt2, msrc2)


def _make_kupd(i):
    last = (i == 2)
    first = (i == 0)

    def body(*refs):
        it = iter(refs)
        agg_ref = next(it)
        h_prev_ref = None if first else next(it)
        pred_ref = next(it)
        r_ref = None if last else next(it)
        u1w, u1b, u2w, u2b = next(it), next(it), next(it), next(it)
        p1w, p1b, p2w, p2b = next(it), next(it), next(it), next(it)
        if not last:
            mw_, mb_ = next(it), next(it)
        outs = list(it)

        agg = jnp.sum(agg_ref[...], axis=0)
        t = _silu(jnp.dot(agg, u1w[...],
                          preferred_element_type=jnp.float32) + u1b[...])
        upd = jnp.dot(t, u2w[...], preferred_element_type=jnp.float32) + u2b[...]
        h = upd if first else h_prev_ref[...] + upd
        h3 = h.reshape(N, NORI, HID)
        hrd = jnp.sum(h3, axis=1) * (1.0 / NORI)
        tp = _silu(jnp.dot(hrd, p1w[...],
                           preferred_element_type=jnp.float32) + p1b[...])
        pred = pred_ref[...] + jnp.dot(tp, p2w[...],
                                       preferred_element_type=jnp.float32) + p2b[...]
        if last:
            outs[0][...] = jnp.sum(pred, axis=0, keepdims=True)
        else:
            h_out, msrc_out, pred_out = outs
            h_out[...] = h
            pred_out[...] = pred
            hm = jnp.dot(h, mw_[...],
                         preferred_element_type=jnp.float32) + mb_[...]
            hm3 = hm.reshape(N, NORI, HID)
            for b in range(NORI):
                acc = hm3[:, 0, :] * r_ref[0 * NORI + b, :][None, :]
                for a in range(1, NORI):
                    acc = acc + hm3[:, a, :] * r_ref[a * NORI + b, :][None, :]
                msrc_out[:, b, :] = acc

    return body


def _run_kupd(i, agg, h_prev, pred, r, p):
    last = (i == 2)
    lp = p["layers"][i]
    pp = p["post"][i]
    args = [agg]
    if i != 0:
        args.append(h_prev)
    args.append(pred)
    if not last:
        args.append(r)
    args += [lp["u1"]["W"], lp["u1"]["b"].reshape(1, -1),
             lp["u2"]["W"], lp["u2"]["b"].reshape(1, -1),
             pp["p1"]["W"], pp["p1"]["b"].reshape(1, -1),
             pp["p2"]["W"], pp["p2"]["b"].reshape(1, -1)]
    if not last:
        ln = p["layers"][i + 1]
        args += [ln["msg"]["W"], ln["msg"]["b"].reshape(1, -1)]
    if last:
        out_shape = [jax.ShapeDtypeStruct((1, OUT), jnp.float32)]
    else:
        out_shape = [jax.ShapeDtypeStruct((N * NORI, HID), jnp.float32),
                     jax.ShapeDtypeStruct((N, NORI, HID), jnp.float32),
                     jax.ShapeDtypeStruct((N, OUT), jnp.float32)]
    return pl.pallas_call(_make_kupd(i), out_shape=out_shape)(*args)


def kernel(pos, x, batch, params):
    px = pos[:, 0]
    py = pos[:, 1]
    pz = pos[:, 2]
    counts2d, src, dst, dxe, dye, dze = _run_compact(px, py, pz)
    counts = counts2d[:, 0]
    xt = _run_kx(counts, dxe.reshape(ECAP, 1), dye.reshape(ECAP, 1),
                 dze.reshape(ECAP, 1), params)
    xt2 = xt.reshape(ECAP, NORI * HID)
    msrc, r = _run_kinit(x, params)
    h = None
    pred = jnp.zeros((N, OUT), jnp.float32)
    for i in range(3):
        aggp = _run_msg_sc(counts2d, src, dst, xt2, msrc.reshape(N, NORI * HID))
        agg = aggp.reshape(NC, N * NORI, HID)
        if i < 2:
            h, msrc, pred = _run_kupd(i, agg, h, pred, r, params)
        else:
            (out,) = _run_kupd(i, agg, h, pred, None, params)
    return out
